# baseline (speedup 1.0000x reference)
import functools

import jax
import jax.numpy as jnp
from jax.experimental import pallas as pl

_B, _C_IN, _N_T = 4, 384, 196
_K, _D = 1024, 64
_P = _B * _N_T


def _vq_body(z_ref, w_ref, emb_ref, out_ref):
    hi = jax.lax.Precision.HIGHEST
    wb = w_ref[...].astype(jnp.bfloat16)
    cols = []
    for b in range(_B):
        zb = z_ref[b].astype(jnp.bfloat16)
        cols.append(jnp.dot(wb, zb, preferred_element_type=jnp.float32))
    ze = jnp.concatenate(cols, axis=1)
    emb = emb_ref[...]
    g = jnp.dot(emb, ze, precision=hi, preferred_element_type=jnp.float32)
    x2 = jnp.sum(ze * ze, axis=0, keepdims=True)
    e2 = jnp.sum(emb * emb, axis=1, keepdims=True)
    d2 = jnp.maximum(x2 - 2.0 * g + e2, 0.0)
    snorm = jnp.sqrt(d2) / (jnp.sqrt(x2) + jnp.sqrt(e2))
    mval = jnp.min(snorm, axis=0, keepdims=True)
    row = jax.lax.broadcasted_iota(jnp.int32, (_K, _P), 0)
    midx = jnp.min(jnp.where(snorm == mval, row, _K), axis=0, keepdims=True)
    onehot = (row == midx).astype(jnp.bfloat16)
    embt = emb.T
    embt_hi = embt.astype(jnp.bfloat16)
    embt_lo = (embt - embt_hi.astype(jnp.float32)).astype(jnp.bfloat16)
    zq = (jnp.dot(embt_hi, onehot, preferred_element_type=jnp.float32)
          + jnp.dot(embt_lo, onehot, preferred_element_type=jnp.float32))
    for b in range(_B):
        out_ref[b] = zq[:, b * _N_T:(b + 1) * _N_T]


@functools.partial(jax.jit, static_argnames=())
def kernel(z, W, emb):
    return pl.pallas_call(
        _vq_body,
        out_shape=jax.ShapeDtypeStruct((_B, _D, _N_T), jnp.float32),
    )(z, W, emb)


# speedup vs baseline: 1.0555x; 1.0555x over previous
import functools

import jax
import jax.numpy as jnp
from jax.experimental import pallas as pl

_B, _C_IN, _N_T = 4, 384, 196
_K, _D = 1024, 64
_P = _B * _N_T


def _vq_body(z_ref, w_ref, emb_ref, out_ref):
    hi = jax.lax.Precision.HIGHEST
    wb = w_ref[...].astype(jnp.bfloat16)
    cols = []
    for b in range(_B):
        zb = z_ref[b].astype(jnp.bfloat16)
        cols.append(jnp.dot(wb, zb, preferred_element_type=jnp.float32))
    ze = jnp.concatenate(cols, axis=1)
    emb = emb_ref[...]
    g = jnp.dot(emb, ze, precision=hi, preferred_element_type=jnp.float32)
    x2 = jnp.sum(ze * ze, axis=0, keepdims=True)
    e2 = jnp.sum(emb * emb, axis=1, keepdims=True)
    d2 = jnp.maximum(x2 - 2.0 * g + e2, 0.0)
    snorm = jnp.sqrt(d2) / (jnp.sqrt(x2) + jnp.sqrt(e2))
    mval = jnp.min(snorm, axis=0, keepdims=True)
    row = jax.lax.broadcasted_iota(jnp.int32, (_K, _P), 0)
    midx = jnp.min(jnp.where(snorm == mval, row, _K), axis=0, keepdims=True)
    onehot = (row == midx).astype(jnp.bfloat16)
    embt = emb.T
    embt_hi = embt.astype(jnp.bfloat16)
    embt_lo = (embt - embt_hi.astype(jnp.float32)).astype(jnp.bfloat16)
    hilo = jnp.concatenate([embt_hi, embt_lo], axis=0)
    r = jnp.dot(hilo, onehot, preferred_element_type=jnp.float32)
    zq = r[:_D] + r[_D:]
    for b in range(_B):
        out_ref[b] = zq[:, b * _N_T:(b + 1) * _N_T]


@functools.partial(jax.jit, static_argnames=())
def kernel(z, W, emb):
    return pl.pallas_call(
        _vq_body,
        out_shape=jax.ShapeDtypeStruct((_B, _D, _N_T), jnp.float32),
    )(z, W, emb)
